# baseline (speedup 1.0000x reference)
import jax
import jax.numpy as jnp
from jax import lax
from jax.experimental import pallas as pl
from jax.experimental.pallas import tpu as pltpu
from jax.experimental.pallas import tpu_sc as plsc

NUM_NODES = 10000
NUM_EDGES = 320000
CH = 128

NC = 2
NS = 16

HN = NUM_NODES // NC
CHUNK = 128
CPT = 160
NCHUNKS = NS * CPT
EPAD = NCHUNKS * CHUNK
DUMP = HN
AROWS = HN + 8
RCHUNK = 40
NRCHUNK = HN // RCHUNK


def _sc_agg_body(emb_hbm, src_hbm, dst_hbm, out_hbm,
                 src_v, dst_v, rows_v, zbuf_v, agg_sh, sem):
    cid = lax.axis_index("c")
    sid = lax.axis_index("s")
    cbase = sid * CPT

    pltpu.sync_copy(src_hbm.at[pl.ds(cbase, CPT)], src_v)
    pltpu.sync_copy(dst_hbm.at[pl.ds(cbase, CPT)], dst_v)

    lo = cid * HN
    def _rrow(r, _):
        def _rcol(j, _):
            d = dst_v[r, pl.ds(j * 16, 16)] - lo
            oob = (d < 0) | (d >= HN)
            dst_v[r, pl.ds(j * 16, 16)] = jnp.where(oob, DUMP, d)
            return 0
        return lax.fori_loop(0, CHUNK // 16, _rcol, 0)
    lax.fori_loop(0, CPT, _rrow, 0)

    def _zrow(i, _):
        def _zcol(j, _):
            zbuf_v[i, pl.ds(j * 16, 16)] = jnp.zeros((16,), jnp.float32)
            return 0
        return lax.fori_loop(0, CH // 16, _zcol, 0)
    lax.fori_loop(0, RCHUNK, _zrow, 0)
    n_mine = 8 - (sid >= NRCHUNK % NS).astype(jnp.int32)
    def _zero(k, _):
        rc = sid + NS * k
        pltpu.sync_copy(zbuf_v, agg_sh.at[pl.ds(rc * RCHUNK, RCHUNK)])
        return 0
    lax.fori_loop(0, n_mine, _zero, 0)
    plsc.subcore_barrier()

    def _step(k, _):
        @pl.when(k < CPT)
        def _():
            pltpu.async_copy(emb_hbm.at[src_v.at[k]],
                             rows_v.at[lax.rem(k, 2)], sem)

        @pl.when(k > 0)
        def _():
            km = k - 1
            pm = lax.rem(km, 2)
            pltpu.make_async_copy(emb_hbm.at[src_v.at[km]],
                                  rows_v.at[pm], sem).wait()
            pltpu.sync_copy(rows_v.at[pm], agg_sh.at[dst_v.at[km]], add=True)
        return 0
    lax.fori_loop(0, CPT + 1, _step, 0)

    plsc.subcore_barrier()

    def _wb(k, _):
        rc = sid + NS * k
        pltpu.sync_copy(agg_sh.at[pl.ds(rc * RCHUNK, RCHUNK)],
                        out_hbm.at[cid, pl.ds(rc * RCHUNK, RCHUNK)])
        return 0
    lax.fori_loop(0, n_mine, _wb, 0)


def _sc_agg(emb, src2d, dst2d):
    mesh = plsc.VectorSubcoreMesh(core_axis_name="c", subcore_axis_name="s",
                                  num_cores=NC, num_subcores=NS)
    fn = pl.kernel(
        _sc_agg_body,
        out_type=jax.ShapeDtypeStruct((NC, HN, CH), jnp.float32),
        mesh=mesh,
        scratch_types=[
            pltpu.VMEM((CPT, CHUNK), jnp.int32),
            pltpu.VMEM((CPT, CHUNK), jnp.int32),
            pltpu.VMEM((2, CHUNK, CH), jnp.float32),
            pltpu.VMEM((RCHUNK, CH), jnp.float32),
            pltpu.VMEM_SHARED((AROWS, CH), jnp.float32),
            pltpu.SemaphoreType.DMA,
        ],
    )
    return fn(emb, src2d, dst2d)


BLK = 1000


def _dense_body(agg_ref, nf_ref, W_ref, U_ref, b_ref, P_ref, bp_ref, out_ref):
    h = jnp.dot(agg_ref[...], W_ref[...], preferred_element_type=jnp.float32)
    h += jnp.dot(nf_ref[...], U_ref[...], preferred_element_type=jnp.float32)
    h = jnp.maximum(h + b_ref[...], 0.0)
    o = jnp.dot(h, P_ref[...], preferred_element_type=jnp.float32)
    o += bp_ref[...]
    m = jnp.max(o, axis=-1, keepdims=True)
    lse = jnp.log(jnp.sum(jnp.exp(o - m), axis=-1, keepdims=True)) + m
    out_ref[...] = o - lse


def _dense(agg, nf, W, U, b, P, bp):
    grid = (NUM_NODES // BLK,)
    return pl.pallas_call(
        _dense_body,
        grid=grid,
        in_specs=[
            pl.BlockSpec((BLK, CH), lambda i: (i, 0)),
            pl.BlockSpec((BLK, CH), lambda i: (i, 0)),
            pl.BlockSpec((CH, CH), lambda i: (0, 0)),
            pl.BlockSpec((CH, CH), lambda i: (0, 0)),
            pl.BlockSpec((1, CH), lambda i: (0, 0)),
            pl.BlockSpec((CH, CH), lambda i: (0, 0)),
            pl.BlockSpec((1, CH), lambda i: (0, 0)),
        ],
        out_specs=pl.BlockSpec((BLK, CH), lambda i: (i, 0)),
        out_shape=jax.ShapeDtypeStruct((NUM_NODES, CH), jnp.float32),
    )(agg, nf, W, U, b, P, bp)


def kernel(node_index, node_feature, edge_index, emb, W, U, b, P, bp):
    npad = EPAD - NUM_EDGES
    src2d = jnp.concatenate(
        [edge_index[0], jnp.zeros((npad,), jnp.int32)]).reshape(-1, CHUNK)
    dst2d = jnp.concatenate(
        [edge_index[1], jnp.full((npad,), NUM_NODES, jnp.int32)]
    ).reshape(-1, CHUNK)
    parts = _sc_agg(emb, src2d, dst2d)
    agg = parts.reshape(NUM_NODES, CH)
    return _dense(agg, node_feature, W, U, b.reshape(1, CH), P,
                  bp.reshape(1, CH))


# speedup vs baseline: 1.3354x; 1.3354x over previous
import jax
import jax.numpy as jnp
from jax import lax
from jax.experimental import pallas as pl
from jax.experimental.pallas import tpu as pltpu
from jax.experimental.pallas import tpu_sc as plsc

NUM_NODES = 10000
NUM_EDGES = 320000
CH = 128

NC = 2
NS = 16
NW = NC * NS

CHUNK = 128
CPT = 80
EPAD = NW * CPT * CHUNK
DUMP = NUM_NODES
AROWS = NUM_NODES + 16
RCHUNK = 80
NRCHUNK = NUM_NODES // RCHUNK


def _sc_agg_body(emb_hbm, src_hbm, dst_hbm, out_hbm,
                 src_v, dst_v, rows_v, zbuf_v, agg_sh, sem):
    cid = lax.axis_index("c")
    sid = lax.axis_index("s")
    wid = sid * NC + cid
    cbase = wid * CPT

    pltpu.sync_copy(src_hbm.at[pl.ds(cbase, CPT)], src_v)
    pltpu.sync_copy(dst_hbm.at[pl.ds(cbase, CPT)], dst_v)

    def _zrow(i, _):
        def _zcol(j, _):
            zbuf_v[i, pl.ds(j * 16, 16)] = jnp.zeros((16,), jnp.float32)
            return 0
        return lax.fori_loop(0, CH // 16, _zcol, 0)
    lax.fori_loop(0, RCHUNK, _zrow, 0)
    n_mine = 8 - (sid >= NRCHUNK % NS).astype(jnp.int32)
    def _zero(k, _):
        rc = sid + NS * k
        pltpu.sync_copy(zbuf_v, agg_sh.at[pl.ds(rc * RCHUNK, RCHUNK)])
        return 0
    lax.fori_loop(0, n_mine, _zero, 0)
    plsc.subcore_barrier()

    def _chunk(k, _):
        pltpu.async_copy(emb_hbm.at[src_v.at[k]], rows_v, sem).wait()
        pltpu.sync_copy(rows_v, agg_sh.at[dst_v.at[k]], add=True)
        return 0
    lax.fori_loop(0, CPT, _chunk, 0)

    plsc.subcore_barrier()

    def _wb(k, _):
        rc = sid + NS * k
        pltpu.sync_copy(agg_sh.at[pl.ds(rc * RCHUNK, RCHUNK)],
                        out_hbm.at[cid, pl.ds(rc * RCHUNK, RCHUNK)])
        return 0
    lax.fori_loop(0, n_mine, _wb, 0)


def _sc_agg(emb, src2d, dst2d):
    mesh = plsc.VectorSubcoreMesh(core_axis_name="c", subcore_axis_name="s",
                                  num_cores=NC, num_subcores=NS)
    fn = pl.kernel(
        _sc_agg_body,
        out_type=jax.ShapeDtypeStruct((NC, NUM_NODES, CH), jnp.float32),
        mesh=mesh,
        scratch_types=[
            pltpu.VMEM((CPT, CHUNK), jnp.int32),
            pltpu.VMEM((CPT, CHUNK), jnp.int32),
            pltpu.VMEM((CHUNK, CH), jnp.float32),
            pltpu.VMEM((RCHUNK, CH), jnp.float32),
            pltpu.VMEM_SHARED((AROWS, CH), jnp.float32),
            pltpu.SemaphoreType.DMA,
        ],
    )
    return fn(emb, src2d, dst2d)


BLK = 1000


def _dense_body(agg_ref, nf_ref, W_ref, U_ref, b_ref, P_ref, bp_ref, out_ref):
    a = agg_ref[0] + agg_ref[1]
    h = jnp.dot(a, W_ref[...], preferred_element_type=jnp.float32)
    h += jnp.dot(nf_ref[...], U_ref[...], preferred_element_type=jnp.float32)
    h = jnp.maximum(h + b_ref[...], 0.0)
    o = jnp.dot(h, P_ref[...], preferred_element_type=jnp.float32)
    o += bp_ref[...]
    m = jnp.max(o, axis=-1, keepdims=True)
    lse = jnp.log(jnp.sum(jnp.exp(o - m), axis=-1, keepdims=True)) + m
    out_ref[...] = o - lse


def _dense(parts, nf, W, U, b, P, bp):
    grid = (NUM_NODES // BLK,)
    return pl.pallas_call(
        _dense_body,
        grid=grid,
        in_specs=[
            pl.BlockSpec((NC, BLK, CH), lambda i: (0, i, 0)),
            pl.BlockSpec((BLK, CH), lambda i: (i, 0)),
            pl.BlockSpec((CH, CH), lambda i: (0, 0)),
            pl.BlockSpec((CH, CH), lambda i: (0, 0)),
            pl.BlockSpec((1, CH), lambda i: (0, 0)),
            pl.BlockSpec((CH, CH), lambda i: (0, 0)),
            pl.BlockSpec((1, CH), lambda i: (0, 0)),
        ],
        out_specs=pl.BlockSpec((BLK, CH), lambda i: (i, 0)),
        out_shape=jax.ShapeDtypeStruct((NUM_NODES, CH), jnp.float32),
    )(parts, nf, W, U, b, P, bp)


def kernel(node_index, node_feature, edge_index, emb, W, U, b, P, bp):
    npad = EPAD - NUM_EDGES
    src2d = jnp.concatenate(
        [edge_index[0], jnp.zeros((npad,), jnp.int32)]).reshape(-1, CHUNK)
    dst2d = jnp.concatenate(
        [edge_index[1], jnp.full((npad,), DUMP, jnp.int32)]).reshape(-1, CHUNK)
    parts = _sc_agg(emb, src2d, dst2d)
    return _dense(parts, node_feature, W, U, b.reshape(1, CH), P,
                  bp.reshape(1, CH))


# speedup vs baseline: 3.2723x; 2.4504x over previous
import jax
import jax.numpy as jnp
from jax import lax
from jax.experimental import pallas as pl
from jax.experimental.pallas import tpu as pltpu
from jax.experimental.pallas import tpu_sc as plsc

NUM_NODES = 10000
NUM_EDGES = 320000
CH = 128

NC = 2
NS = 16
NW = NC * NS

CHUNK = 128
CPT = 80
EPT = CPT * CHUNK
LAST_CPT = (NUM_EDGES - (NW - 1) * EPT) // CHUNK
RCHUNK = 80
NRCHUNK = NUM_NODES // RCHUNK


def _sc_agg_body(emb_hbm, src_hbm, dst_hbm, out_hbm,
                 src_v, dst_v, rows_v, zbuf_v, agg_sh, sem):
    cid = lax.axis_index("c")
    sid = lax.axis_index("s")
    wid = sid * NC + cid
    ebase = wid * EPT
    n_chk = jnp.where(wid == NW - 1, LAST_CPT, CPT)
    sbase = jnp.minimum(ebase, NUM_EDGES - EPT)
    koff = (ebase - sbase) // CHUNK

    pltpu.sync_copy(src_hbm.at[pl.ds(sbase, EPT)], src_v)

    def _ld(k, _):
        pltpu.sync_copy(dst_hbm.at[pl.ds(sbase + k * CHUNK, CHUNK)],
                        dst_v.at[k])
        return 0
    lax.fori_loop(0, CPT, _ld, 0)

    def _zrow(i, _):
        def _zcol(j, _):
            zbuf_v[i, pl.ds(j * 16, 16)] = jnp.zeros((16,), jnp.float32)
            return 0
        return lax.fori_loop(0, CH // 16, _zcol, 0)
    lax.fori_loop(0, RCHUNK, _zrow, 0)
    n_mine = 8 - (sid >= NRCHUNK % NS).astype(jnp.int32)
    def _zero(k, _):
        rc = sid + NS * k
        pltpu.sync_copy(zbuf_v, agg_sh.at[pl.ds(rc * RCHUNK, RCHUNK)])
        return 0
    lax.fori_loop(0, n_mine, _zero, 0)
    plsc.subcore_barrier()

    def _chunk(k, _):
        ks = k + koff
        pltpu.async_copy(emb_hbm.at[src_v.at[pl.ds(ks * CHUNK, CHUNK)]],
                         rows_v, sem).wait()
        pltpu.sync_copy(rows_v, agg_sh.at[dst_v.at[ks]], add=True)
        return 0
    lax.fori_loop(0, n_chk, _chunk, 0)

    plsc.subcore_barrier()

    def _wb(k, _):
        rc = sid + NS * k
        pltpu.sync_copy(agg_sh.at[pl.ds(rc * RCHUNK, RCHUNK)],
                        out_hbm.at[cid, pl.ds(rc * RCHUNK, RCHUNK)])
        return 0
    lax.fori_loop(0, n_mine, _wb, 0)


def _sc_agg(emb, src, dst):
    mesh = plsc.VectorSubcoreMesh(core_axis_name="c", subcore_axis_name="s",
                                  num_cores=NC, num_subcores=NS)
    fn = pl.kernel(
        _sc_agg_body,
        out_type=jax.ShapeDtypeStruct((NC, NUM_NODES, CH), jnp.float32),
        mesh=mesh,
        scratch_types=[
            pltpu.VMEM((EPT,), jnp.int32),
            pltpu.VMEM((CPT, CHUNK), jnp.int32),
            pltpu.VMEM((CHUNK, CH), jnp.float32),
            pltpu.VMEM((RCHUNK, CH), jnp.float32),
            pltpu.VMEM_SHARED((NUM_NODES, CH), jnp.float32),
            pltpu.SemaphoreType.DMA,
        ],
    )
    return fn(emb, src, dst)


BLK = 1000


def _dense_body(agg_ref, nf_ref, W_ref, U_ref, b_ref, P_ref, bp_ref, out_ref):
    a = agg_ref[0] + agg_ref[1]
    h = jnp.dot(a, W_ref[...], preferred_element_type=jnp.float32)
    h += jnp.dot(nf_ref[...], U_ref[...], preferred_element_type=jnp.float32)
    h = jnp.maximum(h + b_ref[...], 0.0)
    o = jnp.dot(h, P_ref[...], preferred_element_type=jnp.float32)
    o += bp_ref[...]
    m = jnp.max(o, axis=-1, keepdims=True)
    lse = jnp.log(jnp.sum(jnp.exp(o - m), axis=-1, keepdims=True)) + m
    out_ref[...] = o - lse


def _dense(parts, nf, W, U, b, P, bp):
    grid = (NUM_NODES // BLK,)
    return pl.pallas_call(
        _dense_body,
        grid=grid,
        in_specs=[
            pl.BlockSpec((NC, BLK, CH), lambda i: (0, i, 0)),
            pl.BlockSpec((BLK, CH), lambda i: (i, 0)),
            pl.BlockSpec((CH, CH), lambda i: (0, 0)),
            pl.BlockSpec((CH, CH), lambda i: (0, 0)),
            pl.BlockSpec((1, CH), lambda i: (0, 0)),
            pl.BlockSpec((CH, CH), lambda i: (0, 0)),
            pl.BlockSpec((1, CH), lambda i: (0, 0)),
        ],
        out_specs=pl.BlockSpec((BLK, CH), lambda i: (i, 0)),
        out_shape=jax.ShapeDtypeStruct((NUM_NODES, CH), jnp.float32),
    )(parts, nf, W, U, b, P, bp)


def kernel(node_index, node_feature, edge_index, emb, W, U, b, P, bp):
    parts = _sc_agg(emb, edge_index[0], edge_index[1])
    return _dense(parts, node_feature, W, U, b.reshape(1, CH), P,
                  bp.reshape(1, CH))


# speedup vs baseline: 3.7818x; 1.1557x over previous
import jax
import jax.numpy as jnp
from jax import lax
from jax.experimental import pallas as pl
from jax.experimental.pallas import tpu as pltpu
from jax.experimental.pallas import tpu_sc as plsc

NUM_NODES = 10000
NUM_EDGES = 320000
CH = 128

NC = 2
NS = 16
NW = NC * NS

CHUNK = 128
CPT = 80
EPT = CPT * CHUNK
LAST_CPT = (NUM_EDGES - (NW - 1) * EPT) // CHUNK
RCHUNK = 80
NRCHUNK = NUM_NODES // RCHUNK


def _sc_agg_body(emb_hbm, src_hbm, dst_hbm, out_hbm,
                 src_v, dst_v, rows_v, zbuf_v, agg_sh, sem):
    cid = lax.axis_index("c")
    sid = lax.axis_index("s")
    wid = sid * NC + cid
    ebase = wid * EPT
    n_chk = jnp.where(wid == NW - 1, LAST_CPT, CPT)
    sbase = jnp.minimum(ebase, NUM_EDGES - EPT)
    koff = (ebase - sbase) // CHUNK

    pltpu.sync_copy(src_hbm.at[pl.ds(sbase, EPT)], src_v)
    pltpu.sync_copy(dst_hbm.at[pl.ds(wid * CPT, CPT)], dst_v)

    def _zrow(i, _):
        def _zcol(j, _):
            zbuf_v[i, pl.ds(j * 16, 16)] = jnp.zeros((16,), jnp.float32)
            return 0
        return lax.fori_loop(0, CH // 16, _zcol, 0)
    lax.fori_loop(0, RCHUNK, _zrow, 0)
    n_mine = 8 - (sid >= NRCHUNK % NS).astype(jnp.int32)
    def _zero(k, _):
        rc = sid + NS * k
        pltpu.sync_copy(zbuf_v, agg_sh.at[pl.ds(rc * RCHUNK, RCHUNK)])
        return 0
    lax.fori_loop(0, n_mine, _zero, 0)
    plsc.subcore_barrier()

    def _chunk(k, _):
        pltpu.async_copy(emb_hbm.at[src_v.at[pl.ds((k + koff) * CHUNK,
                                                   CHUNK)]],
                         rows_v, sem).wait()
        pltpu.sync_copy(rows_v, agg_sh.at[dst_v.at[k]], add=True)
        return 0
    lax.fori_loop(0, n_chk, _chunk, 0)

    plsc.subcore_barrier()

    def _wb(k, _):
        rc = sid + NS * k
        pltpu.sync_copy(agg_sh.at[pl.ds(rc * RCHUNK, RCHUNK)],
                        out_hbm.at[cid, pl.ds(rc * RCHUNK, RCHUNK)])
        return 0
    lax.fori_loop(0, n_mine, _wb, 0)


def _sc_agg(emb, src, dst):
    mesh = plsc.VectorSubcoreMesh(core_axis_name="c", subcore_axis_name="s",
                                  num_cores=NC, num_subcores=NS)
    fn = pl.kernel(
        _sc_agg_body,
        out_type=jax.ShapeDtypeStruct((NC, NUM_NODES, CH), jnp.float32),
        mesh=mesh,
        scratch_types=[
            pltpu.VMEM((EPT,), jnp.int32),
            pltpu.VMEM((CPT, CHUNK), jnp.int32),
            pltpu.VMEM((CHUNK, CH), jnp.float32),
            pltpu.VMEM((RCHUNK, CH), jnp.float32),
            pltpu.VMEM_SHARED((NUM_NODES, CH), jnp.float32),
            pltpu.SemaphoreType.DMA,
        ],
    )
    return fn(emb, src, dst)


BLK = 1000


def _dense_body(agg_ref, nf_ref, W_ref, U_ref, b_ref, P_ref, bp_ref, out_ref):
    a = agg_ref[0] + agg_ref[1]
    h = jnp.dot(a, W_ref[...], preferred_element_type=jnp.float32)
    h += jnp.dot(nf_ref[...], U_ref[...], preferred_element_type=jnp.float32)
    h = jnp.maximum(h + b_ref[...], 0.0)
    o = jnp.dot(h, P_ref[...], preferred_element_type=jnp.float32)
    o += bp_ref[...]
    m = jnp.max(o, axis=-1, keepdims=True)
    lse = jnp.log(jnp.sum(jnp.exp(o - m), axis=-1, keepdims=True)) + m
    out_ref[...] = o - lse


def _dense(parts, nf, W, U, b, P, bp):
    grid = (NUM_NODES // BLK,)
    return pl.pallas_call(
        _dense_body,
        grid=grid,
        in_specs=[
            pl.BlockSpec((NC, BLK, CH), lambda i: (0, i, 0)),
            pl.BlockSpec((BLK, CH), lambda i: (i, 0)),
            pl.BlockSpec((CH, CH), lambda i: (0, 0)),
            pl.BlockSpec((CH, CH), lambda i: (0, 0)),
            pl.BlockSpec((1, CH), lambda i: (0, 0)),
            pl.BlockSpec((CH, CH), lambda i: (0, 0)),
            pl.BlockSpec((1, CH), lambda i: (0, 0)),
        ],
        out_specs=pl.BlockSpec((BLK, CH), lambda i: (i, 0)),
        out_shape=jax.ShapeDtypeStruct((NUM_NODES, CH), jnp.float32),
    )(parts, nf, W, U, b, P, bp)


def kernel(node_index, node_feature, edge_index, emb, W, U, b, P, bp):
    dst2d = jnp.pad(edge_index[1].reshape(NUM_EDGES // CHUNK, CHUNK),
                    ((0, NW * CPT - NUM_EDGES // CHUNK), (0, 0)))
    parts = _sc_agg(emb, edge_index[0], dst2d)
    return _dense(parts, node_feature, W, U, b.reshape(1, CH), P,
                  bp.reshape(1, CH))
